# baseline (speedup 1.0000x reference)
import functools

import jax
import jax.numpy as jnp
from jax import lax
from jax.experimental import pallas as pl
from jax.experimental.pallas import tpu as pltpu
from jax.experimental.pallas import tpu_sc as plsc

N = 10000
E = 320000
D = 128
NC = 2
NS = 16
L = 16
NW = NC * NS
EPW = E // NW
DJ = D // L
UNROLL = 8

CHA = 128
NCHA = EPW // CHA
TAIL = EPW - NCHA * CHA
TBASE = NCHA * CHA

CHB = 64
NCHB = EPW // CHB


def _sc_mesh():
    return plsc.VectorSubcoreMesh(core_axis_name="c", subcore_axis_name="s")


def _dot16(rb, cb, g, lanes):
    def edge_body(k, sv):
        e = g * L + k
        acc = rb[e, pl.ds(0, L)] * cb[e, pl.ds(0, L)]
        for j in range(1, DJ):
            acc = acc + rb[e, pl.ds(j * L, L)] * cb[e, pl.ds(j * L, L)]
        return jnp.where(lanes == k, jnp.sum(acc), sv)

    return lax.fori_loop(0, L, edge_body, jnp.zeros((L,), jnp.float32),
                         unroll=UNROLL)


CH = 80
NCHUNK = EPW // CH


def _scores_body(h_hbm, row3_hbm, col3_hbm, scores_hbm, tmax_hbm,
                 ridx2, cidx2, rbuf0, rbuf1, cbuf0, cbuf1, sbuf, mbuf,
                 rsem0, rsem1, csem0, csem1):
    cid = lax.axis_index("c")
    sid = lax.axis_index("s")
    wid = sid * NC + cid
    ebase = wid * EPW

    lanes = jnp.arange(L, dtype=jnp.int32)
    rbufs = (rbuf0, rbuf1)
    cbufs = (cbuf0, cbuf1)
    rsems = (rsem0, rsem1)
    csems = (csem0, csem1)

    pltpu.sync_copy(row3_hbm.at[wid], ridx2)
    pltpu.sync_copy(col3_hbm.at[wid], cidx2)

    def fire(c, b):
        pltpu.async_copy(h_hbm.at[ridx2.at[c]], rbufs[b], rsems[b])
        pltpu.async_copy(h_hbm.at[cidx2.at[c]], cbufs[b], csems[b])

    def drain(b):
        pltpu.make_async_copy(h_hbm.at[ridx2.at[0]], rbufs[b], rsems[b]).wait()
        pltpu.make_async_copy(h_hbm.at[cidx2.at[0]], cbufs[b], csems[b]).wait()

    def compute_chunk(c, rb, cb, m):
        def group_body(g, m):
            def edge_body(k, sv):
                e = g * L + k
                acc = rb[e, pl.ds(0, L)] * cb[e, pl.ds(0, L)]
                for j in range(1, DJ):
                    acc = acc + rb[e, pl.ds(j * L, L)] * cb[e, pl.ds(j * L, L)]
                return jnp.where(lanes == k, jnp.sum(acc), sv)

            sv = lax.fori_loop(0, L, edge_body, jnp.zeros((L,), jnp.float32),
                               unroll=8)
            sv = jnp.where(sv >= 0.0, sv, 0.2 * sv)
            sbuf[pl.ds(c * CH + g * L, L)] = sv
            return jnp.maximum(m, sv)

        return lax.fori_loop(0, CH // L, group_body, m)

    fire(0, 0)
    fire(1, 1)

    def pair_body(t, m):
        for b in range(2):
            c = 2 * t + b
            drain(b)
            m = compute_chunk(c, rbufs[b], cbufs[b], m)

            @pl.when(c + 2 < NCHUNK)
            def _():
                fire(c + 2, b)
        return m

    m = lax.fori_loop(0, NCHUNK // 2, pair_body,
                      jnp.full((L,), -jnp.inf, jnp.float32))
    drain(0)
    m = compute_chunk(NCHUNK - 1, rbuf0, cbuf0, m)

    mbuf[...] = m
    pltpu.sync_copy(sbuf, scores_hbm.at[pl.ds(ebase, EPW)])
    pltpu.sync_copy(mbuf, tmax_hbm.at[wid])


def _phase_a(h, row3, col3):
    f = pl.kernel(
        _scores_body,
        out_type=(
            jax.ShapeDtypeStruct((E,), jnp.float32),
            jax.ShapeDtypeStruct((NW, L), jnp.float32),
        ),
        mesh=_sc_mesh(),
        compiler_params=pltpu.CompilerParams(needs_layout_passes=False),
        scratch_types=[
            pltpu.VMEM((NCHUNK, CH), jnp.int32),
            pltpu.VMEM((NCHUNK, CH), jnp.int32),
            pltpu.VMEM((CH, D), jnp.float32),
            pltpu.VMEM((CH, D), jnp.float32),
            pltpu.VMEM((CH, D), jnp.float32),
            pltpu.VMEM((CH, D), jnp.float32),
            pltpu.VMEM((EPW,), jnp.float32),
            pltpu.VMEM((L,), jnp.float32),
            pltpu.SemaphoreType.DMA,
            pltpu.SemaphoreType.DMA,
            pltpu.SemaphoreType.DMA,
            pltpu.SemaphoreType.DMA,
        ],
    )
    return f(h, row3, col3)


def _accum_body(h_hbm, row_hbm, col_hbm, scores_hbm, tmax_hbm,
                opart_hbm, zpart_hbm,
                ci0, ci1, ci2, ci3, ci4, ci5,
                ri0, ri1, ri2, ri3, ri4, ri5,
                sk0, sk1, sk2, sk3, sk4, sk5,
                rows0, rows1, rows2, mtbuf, zbuf, acc,
                is0, is1, is2, is3, is4, is5,
                gsem0, gsem1, gsem2, ssem0, ssem1, ssem2):
    cid = lax.axis_index("c")
    sid = lax.axis_index("s")
    wid = sid * NC + cid
    ebase = wid * EPW
    lanes = jnp.arange(L, dtype=jnp.int32)

    cib = (ci0, ci1, ci2, ci3, ci4, ci5)
    rib = (ri0, ri1, ri2, ri3, ri4, ri5)
    skb = (sk0, sk1, sk2, sk3, sk4, sk5)
    isems = (is0, is1, is2, is3, is4, is5)
    rowsb = (rows0, rows1, rows2)
    gsems = (gsem0, gsem1, gsem2)
    ssems = (ssem0, ssem1, ssem2)

    pltpu.sync_copy(tmax_hbm, mtbuf)

    def max_body(k, mv):
        return jnp.maximum(mv, mtbuf[k, pl.ds(0, L)])

    mv = lax.fori_loop(0, NW, max_body, jnp.full((L,), -jnp.inf, jnp.float32))
    m = jnp.max(mv)

    def zrow(e, _):
        for j in range(DJ):
            rows0[e, pl.ds(j * L, L)] = jnp.zeros((L,), jnp.float32)
        return 0

    lax.fori_loop(0, CH, zrow, 0)

    def zchunk(c, _):
        @pl.when(lax.rem(c, NS) == sid)
        def _():
            pltpu.sync_copy(rows0, acc.at[pl.ds(c * CH, CH)])
        return 0

    lax.fori_loop(0, N // CH, zchunk, 0)
    plsc.subcore_barrier()

    def fire_idx(c, i):
        base = ebase + c * CH
        pltpu.async_copy(col_hbm.at[pl.ds(base, CH)], cib[i], isems[i])
        pltpu.async_copy(row_hbm.at[pl.ds(base, CH)], rib[i], isems[i])
        pltpu.async_copy(scores_hbm.at[pl.ds(base, CH)], skb[i], isems[i])

    def drain_idx(i):
        pltpu.make_async_copy(col_hbm.at[pl.ds(0, CH)], cib[i], isems[i]).wait()
        pltpu.make_async_copy(row_hbm.at[pl.ds(0, CH)], rib[i], isems[i]).wait()
        pltpu.make_async_copy(scores_hbm.at[pl.ds(0, CH)], skb[i],
                              isems[i]).wait()

    def fireg(c, b, i):
        pltpu.async_copy(h_hbm.at[cib[i]], rowsb[b], gsems[b])

    def draing(b):
        pltpu.make_async_copy(h_hbm.at[ci0], rowsb[b], gsems[b]).wait()

    def fires(b, i):
        pltpu.async_copy(rowsb[b], acc.at[rib[i]], ssems[b], add=True)

    def drains(b):
        pltpu.make_async_copy(rowsb[b], acc.at[ri0], ssems[b]).wait()

    def compute_chunk(b, i, zacc):
        def pgroup(g, zacc):
            pv = jnp.exp(skb[i][pl.ds(g * L, L)] - m)
            zacc = zacc + pv

            def scale_edge(k, _):
                ps = jnp.sum(jnp.where(lanes == k, pv, 0.0))
                e = g * L + k
                rows = rowsb[b]
                for j in range(DJ):
                    rows[e, pl.ds(j * L, L)] = rows[e, pl.ds(j * L, L)] * ps
                return 0

            lax.fori_loop(0, L, scale_edge, 0, unroll=UNROLL)
            return zacc

        return lax.fori_loop(0, CH // L, pgroup, zacc)

    def step(c, b6, zacc, static_tail):
        b = b6 % 3
        i = b6 % 6
        i2 = (b6 + 2) % 6
        b2 = (b6 + 2) % 3
        draing(b)
        zacc = compute_chunk(b, i, zacc)
        fires(b, i)

        def prefetch_gather():
            drains(b2)
            drain_idx(i2)
            fireg(c + 2, b2, i2)

        if static_tail:
            if static_tail[0] + 2 < NCHUNK:
                prefetch_gather()
            if static_tail[0] + 4 < NCHUNK:
                fire_idx(static_tail[0] + 4, (b6 + 4) % 6)
        else:
            @pl.when(c >= 1)
            def _():
                drains(b2)

            drain_idx(i2)
            fireg(c + 2, b2, i2)
            fire_idx(c + 4, (b6 + 4) % 6)
        return zacc

    for c0 in range(4):
        fire_idx(c0, c0)
    drain_idx(0)
    fireg(0, 0, 0)
    drain_idx(1)
    fireg(1, 1, 1)

    NMAIN = (NCHUNK // 6) * 6

    def sext_body(t, zacc):
        for b6 in range(6):
            zacc = step(6 * t + b6, b6, zacc, None)
        return zacc

    zacc = lax.fori_loop(0, NMAIN // 6, sext_body,
                         jnp.zeros((L,), jnp.float32))
    for c in range(NMAIN, NCHUNK):
        zacc = step(c, c % 6, zacc, (c,))
    drains((NCHUNK - 3) % 3)
    drains((NCHUNK - 2) % 3)
    drains((NCHUNK - 1) % 3)

    zbuf[...] = zacc
    pltpu.sync_copy(zbuf, zpart_hbm.at[wid])

    plsc.subcore_barrier()

    def echunk(c, _):
        @pl.when(lax.rem(c, NS) == sid)
        def _():
            pltpu.sync_copy(acc.at[pl.ds(c * CH, CH)],
                            opart_hbm.at[cid, pl.ds(c * CH, CH)])
        return 0

    lax.fori_loop(0, N // CH, echunk, 0)


def _phase_b(h, row, col, scores, tmax):
    f = pl.kernel(
        _accum_body,
        out_type=(
            jax.ShapeDtypeStruct((NC, N, D), jnp.float32),
            jax.ShapeDtypeStruct((NW, L), jnp.float32),
        ),
        mesh=_sc_mesh(),
        compiler_params=pltpu.CompilerParams(needs_layout_passes=False),
        scratch_types=(
            [pltpu.VMEM((CH,), jnp.int32)] * 6
            + [pltpu.VMEM((CH,), jnp.int32)] * 6
            + [pltpu.VMEM((CH,), jnp.float32)] * 6
            + [pltpu.VMEM((CH, D), jnp.float32)] * 3
            + [pltpu.VMEM((NW, L), jnp.float32), pltpu.VMEM((L,), jnp.float32),
               pltpu.VMEM_SHARED((N, D), jnp.float32)]
            + [pltpu.SemaphoreType.DMA] * 12
        ),
    )
    return f(h, row, col, scores, tmax)


def _mm_relu_body(x_ref, w_ref, b_ref, o_ref):
    y = jnp.dot(x_ref[...], w_ref[...], preferred_element_type=jnp.float32)
    o_ref[...] = jnp.maximum(y + b_ref[...], 0.0)


def _tc_mm_relu(x, w, b):
    return pl.pallas_call(
        _mm_relu_body,
        out_shape=jax.ShapeDtypeStruct((N, D), jnp.float32),
    )(x, w, b.reshape(1, D))


def _comb_body(relu, p_ref, z_ref, w_ref, b_ref, o_ref):
    zinv = 1.0 / jnp.sum(z_ref[...])
    x = (p_ref[0] + p_ref[1]) * zinv
    y = jnp.dot(x, w_ref[...], preferred_element_type=jnp.float32) + b_ref[...]
    if relu:
        y = jnp.maximum(y, 0.0)
    o_ref[...] = y


def _tc_combine_mm(p, z, w, b, relu):
    return pl.pallas_call(
        functools.partial(_comb_body, relu),
        out_shape=jax.ShapeDtypeStruct((N, D), jnp.float32),
    )(p, z, w, b.reshape(1, D))


def kernel(x, edge_index, W0, b0, W1, b1, W2, b2):
    row = edge_index[0]
    col = edge_index[1]
    row3 = row.reshape(NW, NCHUNK, CH)
    col3 = col.reshape(NW, NCHUNK, CH)

    h0 = _tc_mm_relu(x, W0[0], b0[0])
    scores0, tmax0 = _phase_a(h0, row3, col3)
    opart0, zpart0 = _phase_b(h0, row, col, scores0, tmax0)

    h1 = _tc_combine_mm(opart0, zpart0, W1[0], b1[0], relu=True)
    scores1, tmax1 = _phase_a(h1, row3, col3)
    opart1, zpart1 = _phase_b(h1, row, col, scores1, tmax1)

    return _tc_combine_mm(opart1, zpart1, W2, b2, relu=False)


# speedup vs baseline: 1.1006x; 1.1006x over previous
import functools

import jax
import jax.numpy as jnp
from jax import lax
from jax.experimental import pallas as pl
from jax.experimental.pallas import tpu as pltpu
from jax.experimental.pallas import tpu_sc as plsc

N = 10000
E = 320000
D = 128
NC = 2
NS = 16
L = 16
NW = NC * NS
EPW = E // NW
DJ = D // L
UNROLL = 8

CHA = 128
NCHA = EPW // CHA
TAIL = EPW - NCHA * CHA
TBASE = NCHA * CHA

CHB = 64
NCHB = EPW // CHB


def _sc_mesh():
    return plsc.VectorSubcoreMesh(core_axis_name="c", subcore_axis_name="s")


def _dot16(rb, cb, g, lanes):
    def edge_body(k, sv):
        e = g * L + k
        acc = rb[e, pl.ds(0, L)] * cb[e, pl.ds(0, L)]
        for j in range(1, DJ):
            acc = acc + rb[e, pl.ds(j * L, L)] * cb[e, pl.ds(j * L, L)]
        return jnp.where(lanes == k, jnp.sum(acc), sv)

    return lax.fori_loop(0, L, edge_body, jnp.zeros((L,), jnp.float32),
                         unroll=UNROLL)


CH = 80
NCHUNK = EPW // CH


def _scores_body(h_hbm, row3_hbm, col3_hbm, scores_hbm, tmax_hbm,
                 ridx2, cidx2, rbuf0, rbuf1, cbuf0, cbuf1, sbuf, mbuf,
                 rsem0, rsem1, csem0, csem1):
    cid = lax.axis_index("c")
    sid = lax.axis_index("s")
    wid = sid * NC + cid
    ebase = wid * EPW

    lanes = jnp.arange(L, dtype=jnp.int32)
    rbufs = (rbuf0, rbuf1)
    cbufs = (cbuf0, cbuf1)
    rsems = (rsem0, rsem1)
    csems = (csem0, csem1)

    pltpu.sync_copy(row3_hbm.at[wid], ridx2)
    pltpu.sync_copy(col3_hbm.at[wid], cidx2)

    def fire(c, b):
        pltpu.async_copy(h_hbm.at[ridx2.at[c]], rbufs[b], rsems[b])
        pltpu.async_copy(h_hbm.at[cidx2.at[c]], cbufs[b], csems[b])

    def drain(b):
        pltpu.make_async_copy(h_hbm.at[ridx2.at[0]], rbufs[b], rsems[b]).wait()
        pltpu.make_async_copy(h_hbm.at[cidx2.at[0]], cbufs[b], csems[b]).wait()

    def compute_chunk(c, rb, cb, m):
        def group_body(g, m):
            def edge_body(k, sv):
                e = g * L + k
                acc = rb[e, pl.ds(0, L)] * cb[e, pl.ds(0, L)]
                for j in range(1, DJ):
                    acc = acc + rb[e, pl.ds(j * L, L)] * cb[e, pl.ds(j * L, L)]
                return jnp.where(lanes == k, jnp.sum(acc), sv)

            sv = lax.fori_loop(0, L, edge_body, jnp.zeros((L,), jnp.float32),
                               unroll=4)
            sv = jnp.where(sv >= 0.0, sv, 0.2 * sv)
            sbuf[pl.ds(c * CH + g * L, L)] = sv
            return jnp.maximum(m, sv)

        return lax.fori_loop(0, CH // L, group_body, m)

    fire(0, 0)
    fire(1, 1)

    def pair_body(t, m):
        for b in range(2):
            c = 2 * t + b
            drain(b)
            m = compute_chunk(c, rbufs[b], cbufs[b], m)

            @pl.when(c + 2 < NCHUNK)
            def _():
                fire(c + 2, b)
        return m

    m = lax.fori_loop(0, NCHUNK // 2, pair_body,
                      jnp.full((L,), -jnp.inf, jnp.float32))
    drain(0)
    m = compute_chunk(NCHUNK - 1, rbuf0, cbuf0, m)

    mbuf[...] = m
    pltpu.sync_copy(sbuf, scores_hbm.at[pl.ds(ebase, EPW)])
    pltpu.sync_copy(mbuf, tmax_hbm.at[wid])


def _phase_a(h, row3, col3):
    f = pl.kernel(
        _scores_body,
        out_type=(
            jax.ShapeDtypeStruct((E,), jnp.float32),
            jax.ShapeDtypeStruct((NW, L), jnp.float32),
        ),
        mesh=_sc_mesh(),
        compiler_params=pltpu.CompilerParams(needs_layout_passes=False),
        scratch_types=[
            pltpu.VMEM((NCHUNK, CH), jnp.int32),
            pltpu.VMEM((NCHUNK, CH), jnp.int32),
            pltpu.VMEM((CH, D), jnp.float32),
            pltpu.VMEM((CH, D), jnp.float32),
            pltpu.VMEM((CH, D), jnp.float32),
            pltpu.VMEM((CH, D), jnp.float32),
            pltpu.VMEM((EPW,), jnp.float32),
            pltpu.VMEM((L,), jnp.float32),
            pltpu.SemaphoreType.DMA,
            pltpu.SemaphoreType.DMA,
            pltpu.SemaphoreType.DMA,
            pltpu.SemaphoreType.DMA,
        ],
    )
    return f(h, row3, col3)


def _accum_body(h_hbm, row_hbm, col_hbm, scores_hbm, tmax_hbm,
                opart_hbm, zpart_hbm,
                ci0, ci1, ci2, ci3, ci4, ci5,
                ri0, ri1, ri2, ri3, ri4, ri5,
                sk0, sk1, sk2, sk3, sk4, sk5,
                rows0, rows1, rows2, mtbuf, zbuf, acc,
                is0, is1, is2, is3, is4, is5,
                gsem0, gsem1, gsem2, ssem0, ssem1, ssem2):
    cid = lax.axis_index("c")
    sid = lax.axis_index("s")
    wid = sid * NC + cid
    ebase = wid * EPW
    lanes = jnp.arange(L, dtype=jnp.int32)

    cib = (ci0, ci1, ci2, ci3, ci4, ci5)
    rib = (ri0, ri1, ri2, ri3, ri4, ri5)
    skb = (sk0, sk1, sk2, sk3, sk4, sk5)
    isems = (is0, is1, is2, is3, is4, is5)
    rowsb = (rows0, rows1, rows2)
    gsems = (gsem0, gsem1, gsem2)
    ssems = (ssem0, ssem1, ssem2)

    pltpu.sync_copy(tmax_hbm, mtbuf)

    def max_body(k, mv):
        return jnp.maximum(mv, mtbuf[k, pl.ds(0, L)])

    mv = lax.fori_loop(0, NW, max_body, jnp.full((L,), -jnp.inf, jnp.float32))
    m = jnp.max(mv)

    def zrow(e, _):
        for j in range(DJ):
            rows0[e, pl.ds(j * L, L)] = jnp.zeros((L,), jnp.float32)
        return 0

    lax.fori_loop(0, CH, zrow, 0)

    def zchunk(c, _):
        @pl.when(lax.rem(c, NS) == sid)
        def _():
            pltpu.sync_copy(rows0, acc.at[pl.ds(c * CH, CH)])
        return 0

    lax.fori_loop(0, N // CH, zchunk, 0)
    plsc.subcore_barrier()

    def fire_idx(c, i):
        base = ebase + c * CH
        pltpu.async_copy(col_hbm.at[pl.ds(base, CH)], cib[i], isems[i])
        pltpu.async_copy(row_hbm.at[pl.ds(base, CH)], rib[i], isems[i])
        pltpu.async_copy(scores_hbm.at[pl.ds(base, CH)], skb[i], isems[i])

    def drain_idx(i):
        pltpu.make_async_copy(col_hbm.at[pl.ds(0, CH)], cib[i], isems[i]).wait()
        pltpu.make_async_copy(row_hbm.at[pl.ds(0, CH)], rib[i], isems[i]).wait()
        pltpu.make_async_copy(scores_hbm.at[pl.ds(0, CH)], skb[i],
                              isems[i]).wait()

    def fireg(c, b, i):
        pltpu.async_copy(h_hbm.at[cib[i]], rowsb[b], gsems[b])

    def draing(b):
        pltpu.make_async_copy(h_hbm.at[ci0], rowsb[b], gsems[b]).wait()

    def fires(b, i):
        pltpu.async_copy(rowsb[b], acc.at[rib[i]], ssems[b], add=True)

    def drains(b):
        pltpu.make_async_copy(rowsb[b], acc.at[ri0], ssems[b]).wait()

    def compute_chunk(b, i, zacc):
        def pgroup(g, zacc):
            pv = jnp.exp(skb[i][pl.ds(g * L, L)] - m)
            zacc = zacc + pv

            def scale_edge(k, _):
                ps = jnp.sum(jnp.where(lanes == k, pv, 0.0))
                e = g * L + k
                rows = rowsb[b]
                for j in range(DJ):
                    rows[e, pl.ds(j * L, L)] = rows[e, pl.ds(j * L, L)] * ps
                return 0

            lax.fori_loop(0, L, scale_edge, 0, unroll=4)
            return zacc

        return lax.fori_loop(0, CH // L, pgroup, zacc)

    def step(c, b6, zacc, static_tail):
        b = b6 % 3
        i = b6 % 6
        i2 = (b6 + 2) % 6
        b2 = (b6 + 2) % 3
        draing(b)
        zacc = compute_chunk(b, i, zacc)
        fires(b, i)

        def prefetch_gather():
            drains(b2)
            drain_idx(i2)
            fireg(c + 2, b2, i2)

        if static_tail:
            if static_tail[0] + 2 < NCHUNK:
                prefetch_gather()
            if static_tail[0] + 4 < NCHUNK:
                fire_idx(static_tail[0] + 4, (b6 + 4) % 6)
        else:
            @pl.when(c >= 1)
            def _():
                drains(b2)

            drain_idx(i2)
            fireg(c + 2, b2, i2)
            fire_idx(c + 4, (b6 + 4) % 6)
        return zacc

    for c0 in range(4):
        fire_idx(c0, c0)
    drain_idx(0)
    fireg(0, 0, 0)
    drain_idx(1)
    fireg(1, 1, 1)

    NMAIN = (NCHUNK // 6) * 6

    def sext_body(t, zacc):
        for b6 in range(6):
            zacc = step(6 * t + b6, b6, zacc, None)
        return zacc

    zacc = lax.fori_loop(0, NMAIN // 6, sext_body,
                         jnp.zeros((L,), jnp.float32))
    for c in range(NMAIN, NCHUNK):
        zacc = step(c, c % 6, zacc, (c,))
    drains((NCHUNK - 3) % 3)
    drains((NCHUNK - 2) % 3)
    drains((NCHUNK - 1) % 3)

    zbuf[...] = zacc
    pltpu.sync_copy(zbuf, zpart_hbm.at[wid])

    plsc.subcore_barrier()

    def echunk(c, _):
        @pl.when(lax.rem(c, NS) == sid)
        def _():
            pltpu.sync_copy(acc.at[pl.ds(c * CH, CH)],
                            opart_hbm.at[cid, pl.ds(c * CH, CH)])
        return 0

    lax.fori_loop(0, N // CH, echunk, 0)


def _phase_b(h, row, col, scores, tmax):
    f = pl.kernel(
        _accum_body,
        out_type=(
            jax.ShapeDtypeStruct((NC, N, D), jnp.float32),
            jax.ShapeDtypeStruct((NW, L), jnp.float32),
        ),
        mesh=_sc_mesh(),
        compiler_params=pltpu.CompilerParams(needs_layout_passes=False),
        scratch_types=(
            [pltpu.VMEM((CH,), jnp.int32)] * 6
            + [pltpu.VMEM((CH,), jnp.int32)] * 6
            + [pltpu.VMEM((CH,), jnp.float32)] * 6
            + [pltpu.VMEM((CH, D), jnp.float32)] * 3
            + [pltpu.VMEM((NW, L), jnp.float32), pltpu.VMEM((L,), jnp.float32),
               pltpu.VMEM_SHARED((N, D), jnp.float32)]
            + [pltpu.SemaphoreType.DMA] * 12
        ),
    )
    return f(h, row, col, scores, tmax)


def _mm_relu_body(x_ref, w_ref, b_ref, o_ref):
    y = jnp.dot(x_ref[...], w_ref[...], preferred_element_type=jnp.float32)
    o_ref[...] = jnp.maximum(y + b_ref[...], 0.0)


def _tc_mm_relu(x, w, b):
    return pl.pallas_call(
        _mm_relu_body,
        out_shape=jax.ShapeDtypeStruct((N, D), jnp.float32),
    )(x, w, b.reshape(1, D))


def _comb_body(relu, p_ref, z_ref, w_ref, b_ref, o_ref):
    zinv = 1.0 / jnp.sum(z_ref[...])
    x = (p_ref[0] + p_ref[1]) * zinv
    y = jnp.dot(x, w_ref[...], preferred_element_type=jnp.float32) + b_ref[...]
    if relu:
        y = jnp.maximum(y, 0.0)
    o_ref[...] = y


def _tc_combine_mm(p, z, w, b, relu):
    return pl.pallas_call(
        functools.partial(_comb_body, relu),
        out_shape=jax.ShapeDtypeStruct((N, D), jnp.float32),
    )(p, z, w, b.reshape(1, D))


def kernel(x, edge_index, W0, b0, W1, b1, W2, b2):
    row = edge_index[0]
    col = edge_index[1]
    row3 = row.reshape(NW, NCHUNK, CH)
    col3 = col.reshape(NW, NCHUNK, CH)

    h0 = _tc_mm_relu(x, W0[0], b0[0])
    scores0, tmax0 = _phase_a(h0, row3, col3)
    opart0, zpart0 = _phase_b(h0, row, col, scores0, tmax0)

    h1 = _tc_combine_mm(opart0, zpart0, W1[0], b1[0], relu=True)
    scores1, tmax1 = _phase_a(h1, row3, col3)
    opart1, zpart1 = _phase_b(h1, row, col, scores1, tmax1)

    return _tc_combine_mm(opart1, zpart1, W2, b2, relu=False)


# speedup vs baseline: 1.1055x; 1.0044x over previous
import functools

import jax
import jax.numpy as jnp
from jax import lax
from jax.experimental import pallas as pl
from jax.experimental.pallas import tpu as pltpu
from jax.experimental.pallas import tpu_sc as plsc

N = 10000
E = 320000
D = 128
NC = 2
NS = 16
L = 16
NW = NC * NS
EPW = E // NW
DJ = D // L
UNROLL = 8

CHA = 128
NCHA = EPW // CHA
TAIL = EPW - NCHA * CHA
TBASE = NCHA * CHA

CHB = 64
NCHB = EPW // CHB


def _sc_mesh():
    return plsc.VectorSubcoreMesh(core_axis_name="c", subcore_axis_name="s")


def _dot16(rb, cb, g, lanes):
    def edge_body(k, sv):
        e = g * L + k
        acc = rb[e, pl.ds(0, L)] * cb[e, pl.ds(0, L)]
        for j in range(1, DJ):
            acc = acc + rb[e, pl.ds(j * L, L)] * cb[e, pl.ds(j * L, L)]
        return jnp.where(lanes == k, jnp.sum(acc), sv)

    return lax.fori_loop(0, L, edge_body, jnp.zeros((L,), jnp.float32),
                         unroll=UNROLL)


CH = 80
NCHUNK = EPW // CH


def _scores_body(h_hbm, row3_hbm, col3_hbm, scores_hbm, tmax_hbm,
                 ridx2, cidx2, rbuf0, rbuf1, cbuf0, cbuf1, sbuf, mbuf,
                 rsem0, rsem1, csem0, csem1):
    cid = lax.axis_index("c")
    sid = lax.axis_index("s")
    wid = sid * NC + cid
    ebase = wid * EPW

    lanes = jnp.arange(L, dtype=jnp.int32)
    rbufs = (rbuf0, rbuf1)
    cbufs = (cbuf0, cbuf1)
    rsems = (rsem0, rsem1)
    csems = (csem0, csem1)

    pltpu.sync_copy(row3_hbm.at[wid], ridx2)
    pltpu.sync_copy(col3_hbm.at[wid], cidx2)

    def fire(c, b):
        pltpu.async_copy(h_hbm.at[ridx2.at[c]], rbufs[b], rsems[b])
        pltpu.async_copy(h_hbm.at[cidx2.at[c]], cbufs[b], csems[b])

    def drain(b):
        pltpu.make_async_copy(h_hbm.at[ridx2.at[0]], rbufs[b], rsems[b]).wait()
        pltpu.make_async_copy(h_hbm.at[cidx2.at[0]], cbufs[b], csems[b]).wait()

    def compute_chunk(c, rb, cb, m):
        def group_body(g, m):
            def edge_body(k, sv):
                e = g * L + k
                acc = rb[e, pl.ds(0, L)] * cb[e, pl.ds(0, L)]
                for j in range(1, DJ):
                    acc = acc + rb[e, pl.ds(j * L, L)] * cb[e, pl.ds(j * L, L)]
                return jnp.where(lanes == k, jnp.sum(acc), sv)

            sv = lax.fori_loop(0, L, edge_body, jnp.zeros((L,), jnp.float32),
                               unroll=2)
            sv = jnp.where(sv >= 0.0, sv, 0.2 * sv)
            sbuf[pl.ds(c * CH + g * L, L)] = sv
            return jnp.maximum(m, sv)

        return lax.fori_loop(0, CH // L, group_body, m)

    fire(0, 0)
    fire(1, 1)

    def pair_body(t, m):
        for b in range(2):
            c = 2 * t + b
            drain(b)
            m = compute_chunk(c, rbufs[b], cbufs[b], m)

            @pl.when(c + 2 < NCHUNK)
            def _():
                fire(c + 2, b)
        return m

    m = lax.fori_loop(0, NCHUNK // 2, pair_body,
                      jnp.full((L,), -jnp.inf, jnp.float32))
    drain(0)
    m = compute_chunk(NCHUNK - 1, rbuf0, cbuf0, m)

    mbuf[...] = m
    pltpu.sync_copy(sbuf, scores_hbm.at[pl.ds(ebase, EPW)])
    pltpu.sync_copy(mbuf, tmax_hbm.at[wid])


def _phase_a(h, row3, col3):
    f = pl.kernel(
        _scores_body,
        out_type=(
            jax.ShapeDtypeStruct((E,), jnp.float32),
            jax.ShapeDtypeStruct((NW, L), jnp.float32),
        ),
        mesh=_sc_mesh(),
        compiler_params=pltpu.CompilerParams(needs_layout_passes=False),
        scratch_types=[
            pltpu.VMEM((NCHUNK, CH), jnp.int32),
            pltpu.VMEM((NCHUNK, CH), jnp.int32),
            pltpu.VMEM((CH, D), jnp.float32),
            pltpu.VMEM((CH, D), jnp.float32),
            pltpu.VMEM((CH, D), jnp.float32),
            pltpu.VMEM((CH, D), jnp.float32),
            pltpu.VMEM((EPW,), jnp.float32),
            pltpu.VMEM((L,), jnp.float32),
            pltpu.SemaphoreType.DMA,
            pltpu.SemaphoreType.DMA,
            pltpu.SemaphoreType.DMA,
            pltpu.SemaphoreType.DMA,
        ],
    )
    return f(h, row3, col3)


def _accum_body(h_hbm, row_hbm, col_hbm, scores_hbm, tmax_hbm,
                opart_hbm, zpart_hbm,
                ci0, ci1, ci2, ci3, ci4, ci5,
                ri0, ri1, ri2, ri3, ri4, ri5,
                sk0, sk1, sk2, sk3, sk4, sk5,
                rows0, rows1, rows2, mtbuf, zbuf, acc,
                is0, is1, is2, is3, is4, is5,
                gsem0, gsem1, gsem2, ssem0, ssem1, ssem2):
    cid = lax.axis_index("c")
    sid = lax.axis_index("s")
    wid = sid * NC + cid
    ebase = wid * EPW
    lanes = jnp.arange(L, dtype=jnp.int32)

    cib = (ci0, ci1, ci2, ci3, ci4, ci5)
    rib = (ri0, ri1, ri2, ri3, ri4, ri5)
    skb = (sk0, sk1, sk2, sk3, sk4, sk5)
    isems = (is0, is1, is2, is3, is4, is5)
    rowsb = (rows0, rows1, rows2)
    gsems = (gsem0, gsem1, gsem2)
    ssems = (ssem0, ssem1, ssem2)

    pltpu.sync_copy(tmax_hbm, mtbuf)

    def max_body(k, mv):
        return jnp.maximum(mv, mtbuf[k, pl.ds(0, L)])

    mv = lax.fori_loop(0, NW, max_body, jnp.full((L,), -jnp.inf, jnp.float32))
    m = jnp.max(mv)

    def zrow(e, _):
        for j in range(DJ):
            rows0[e, pl.ds(j * L, L)] = jnp.zeros((L,), jnp.float32)
        return 0

    lax.fori_loop(0, CH, zrow, 0)

    def zchunk(c, _):
        @pl.when(lax.rem(c, NS) == sid)
        def _():
            pltpu.sync_copy(rows0, acc.at[pl.ds(c * CH, CH)])
        return 0

    lax.fori_loop(0, N // CH, zchunk, 0)
    plsc.subcore_barrier()

    def fire_idx(c, i):
        base = ebase + c * CH
        pltpu.async_copy(col_hbm.at[pl.ds(base, CH)], cib[i], isems[i])
        pltpu.async_copy(row_hbm.at[pl.ds(base, CH)], rib[i], isems[i])
        pltpu.async_copy(scores_hbm.at[pl.ds(base, CH)], skb[i], isems[i])

    def drain_idx(i):
        pltpu.make_async_copy(col_hbm.at[pl.ds(0, CH)], cib[i], isems[i]).wait()
        pltpu.make_async_copy(row_hbm.at[pl.ds(0, CH)], rib[i], isems[i]).wait()
        pltpu.make_async_copy(scores_hbm.at[pl.ds(0, CH)], skb[i],
                              isems[i]).wait()

    def fireg(c, b, i):
        pltpu.async_copy(h_hbm.at[cib[i]], rowsb[b], gsems[b])

    def draing(b):
        pltpu.make_async_copy(h_hbm.at[ci0], rowsb[b], gsems[b]).wait()

    def fires(b, i):
        pltpu.async_copy(rowsb[b], acc.at[rib[i]], ssems[b], add=True)

    def drains(b):
        pltpu.make_async_copy(rowsb[b], acc.at[ri0], ssems[b]).wait()

    def compute_chunk(b, i, zacc):
        def pgroup(g, zacc):
            pv = jnp.exp(skb[i][pl.ds(g * L, L)] - m)
            zacc = zacc + pv

            def scale_edge(k, _):
                ps = jnp.sum(jnp.where(lanes == k, pv, 0.0))
                e = g * L + k
                rows = rowsb[b]
                for j in range(DJ):
                    rows[e, pl.ds(j * L, L)] = rows[e, pl.ds(j * L, L)] * ps
                return 0

            lax.fori_loop(0, L, scale_edge, 0, unroll=2)
            return zacc

        return lax.fori_loop(0, CH // L, pgroup, zacc)

    def step(c, b6, zacc, static_tail):
        b = b6 % 3
        i = b6 % 6
        i2 = (b6 + 2) % 6
        b2 = (b6 + 2) % 3
        draing(b)
        zacc = compute_chunk(b, i, zacc)
        fires(b, i)

        def prefetch_gather():
            drains(b2)
            drain_idx(i2)
            fireg(c + 2, b2, i2)

        if static_tail:
            if static_tail[0] + 2 < NCHUNK:
                prefetch_gather()
            if static_tail[0] + 4 < NCHUNK:
                fire_idx(static_tail[0] + 4, (b6 + 4) % 6)
        else:
            @pl.when(c >= 1)
            def _():
                drains(b2)

            drain_idx(i2)
            fireg(c + 2, b2, i2)
            fire_idx(c + 4, (b6 + 4) % 6)
        return zacc

    for c0 in range(4):
        fire_idx(c0, c0)
    drain_idx(0)
    fireg(0, 0, 0)
    drain_idx(1)
    fireg(1, 1, 1)

    NMAIN = (NCHUNK // 6) * 6

    def sext_body(t, zacc):
        for b6 in range(6):
            zacc = step(6 * t + b6, b6, zacc, None)
        return zacc

    zacc = lax.fori_loop(0, NMAIN // 6, sext_body,
                         jnp.zeros((L,), jnp.float32))
    for c in range(NMAIN, NCHUNK):
        zacc = step(c, c % 6, zacc, (c,))
    drains((NCHUNK - 3) % 3)
    drains((NCHUNK - 2) % 3)
    drains((NCHUNK - 1) % 3)

    zbuf[...] = zacc
    pltpu.sync_copy(zbuf, zpart_hbm.at[wid])

    plsc.subcore_barrier()

    def echunk(c, _):
        @pl.when(lax.rem(c, NS) == sid)
        def _():
            pltpu.sync_copy(acc.at[pl.ds(c * CH, CH)],
                            opart_hbm.at[cid, pl.ds(c * CH, CH)])
        return 0

    lax.fori_loop(0, N // CH, echunk, 0)


def _phase_b(h, row, col, scores, tmax):
    f = pl.kernel(
        _accum_body,
        out_type=(
            jax.ShapeDtypeStruct((NC, N, D), jnp.float32),
            jax.ShapeDtypeStruct((NW, L), jnp.float32),
        ),
        mesh=_sc_mesh(),
        compiler_params=pltpu.CompilerParams(needs_layout_passes=False),
        scratch_types=(
            [pltpu.VMEM((CH,), jnp.int32)] * 6
            + [pltpu.VMEM((CH,), jnp.int32)] * 6
            + [pltpu.VMEM((CH,), jnp.float32)] * 6
            + [pltpu.VMEM((CH, D), jnp.float32)] * 3
            + [pltpu.VMEM((NW, L), jnp.float32), pltpu.VMEM((L,), jnp.float32),
               pltpu.VMEM_SHARED((N, D), jnp.float32)]
            + [pltpu.SemaphoreType.DMA] * 12
        ),
    )
    return f(h, row, col, scores, tmax)


def _mm_relu_body(x_ref, w_ref, b_ref, o_ref):
    y = jnp.dot(x_ref[...], w_ref[...], preferred_element_type=jnp.float32)
    o_ref[...] = jnp.maximum(y + b_ref[...], 0.0)


def _tc_mm_relu(x, w, b):
    return pl.pallas_call(
        _mm_relu_body,
        out_shape=jax.ShapeDtypeStruct((N, D), jnp.float32),
    )(x, w, b.reshape(1, D))


def _comb_body(relu, p_ref, z_ref, w_ref, b_ref, o_ref):
    zinv = 1.0 / jnp.sum(z_ref[...])
    x = (p_ref[0] + p_ref[1]) * zinv
    y = jnp.dot(x, w_ref[...], preferred_element_type=jnp.float32) + b_ref[...]
    if relu:
        y = jnp.maximum(y, 0.0)
    o_ref[...] = y


def _tc_combine_mm(p, z, w, b, relu):
    return pl.pallas_call(
        functools.partial(_comb_body, relu),
        out_shape=jax.ShapeDtypeStruct((N, D), jnp.float32),
    )(p, z, w, b.reshape(1, D))


def kernel(x, edge_index, W0, b0, W1, b1, W2, b2):
    row = edge_index[0]
    col = edge_index[1]
    row3 = row.reshape(NW, NCHUNK, CH)
    col3 = col.reshape(NW, NCHUNK, CH)

    h0 = _tc_mm_relu(x, W0[0], b0[0])
    scores0, tmax0 = _phase_a(h0, row3, col3)
    opart0, zpart0 = _phase_b(h0, row, col, scores0, tmax0)

    h1 = _tc_combine_mm(opart0, zpart0, W1[0], b1[0], relu=True)
    scores1, tmax1 = _phase_a(h1, row3, col3)
    opart1, zpart1 = _phase_b(h1, row, col, scores1, tmax1)

    return _tc_combine_mm(opart1, zpart1, W2, b2, relu=False)
